# baseline (speedup 1.0000x reference)
import functools

import jax
import jax.numpy as jnp
from jax import lax
from jax.experimental import pallas as pl
from jax.experimental.pallas import tpu as pltpu
from jax.experimental.pallas import tpu_sc as plsc

NC = 2
NS = 16
NW = NC * NS
C = 128
D = 128
SB = 8


def _make_deg_kernel(n_pad, nc):
    mesh = plsc.VectorSubcoreMesh(core_axis_name="c", subcore_axis_name="s")
    rpt = n_pad // NS

    @functools.partial(
        pl.kernel,
        mesh=mesh,
        out_type=jax.ShapeDtypeStruct((NC, n_pad, D), jnp.float32),
        scratch_types=[
            pltpu.VMEM((nc, C), jnp.int32),
            pltpu.VMEM((C, D), jnp.float32),
            pltpu.VMEM_SHARED((n_pad, D), jnp.float32),
        ],
    )
    def k(dst_hbm, ones_hbm, zeros_hbm, out_hbm, dst_v, ones_v, acc):
        cid = lax.axis_index("c")
        sid = lax.axis_index("s")
        wid = sid * NC + cid
        pltpu.sync_copy(zeros_hbm.at[pl.ds(sid * rpt, rpt)],
                        acc.at[pl.ds(sid * rpt, rpt)])
        pltpu.sync_copy(dst_hbm.at[wid], dst_v)
        pltpu.sync_copy(ones_hbm, ones_v)
        plsc.subcore_barrier()

        def body(j, carry):
            pltpu.sync_copy(ones_v, acc.at[dst_v.at[j]], add=True)
            return carry

        lax.fori_loop(0, nc, body, 0)
        plsc.subcore_barrier()
        pltpu.sync_copy(acc.at[pl.ds(sid * rpt, rpt)],
                        out_hbm.at[cid, pl.ds(sid * rpt, rpt)])

    return k


def _make_scatter_kernel(n_pad, nc):
    mesh = plsc.VectorSubcoreMesh(core_axis_name="c", subcore_axis_name="s")
    rpt = n_pad // NS
    nsup = nc // SB

    @functools.partial(
        pl.kernel,
        mesh=mesh,
        out_type=jax.ShapeDtypeStruct((NC, n_pad, D), jnp.float32),
        scratch_types=[
            pltpu.VMEM((2, SB, C), jnp.int32),
            pltpu.VMEM((2, SB, C), jnp.int32),
            pltpu.VMEM((C, D), jnp.float32),
            pltpu.VMEM((C, D), jnp.float32),
            pltpu.VMEM_SHARED((n_pad, D), jnp.float32),
            pltpu.SemaphoreType.DMA,
            pltpu.SemaphoreType.DMA,
            pltpu.SemaphoreType.DMA,
            pltpu.SemaphoreType.DMA,
        ],
    )
    def k(hs_hbm, src_hbm, dst_hbm, zeros_hbm, out_hbm,
          src_i, dst_i, buf0, buf1, acc, g0, g1, is0, is1):
        cid = lax.axis_index("c")
        sid = lax.axis_index("s")
        wid = sid * NC + cid
        pltpu.sync_copy(zeros_hbm.at[pl.ds(sid * rpt, rpt)],
                        acc.at[pl.ds(sid * rpt, rpt)])
        pltpu.sync_copy(src_hbm.at[wid, pl.ds(0, SB)], src_i.at[0])
        pltpu.sync_copy(dst_hbm.at[wid, pl.ds(0, SB)], dst_i.at[0])
        plsc.subcore_barrier()

        bufs = (buf0, buf1)
        gsems = (g0, g1)
        isems = (is0, is1)

        pltpu.async_copy(hs_hbm.at[src_i.at[0, 0]], buf0, g0)
        pltpu.async_copy(hs_hbm.at[src_i.at[0, 1]], buf1, g1)

        def emit_super(m, b):
            nxt = 1 - b
            more = m + 1 < nsup

            @pl.when(more)
            def _():
                pltpu.async_copy(src_hbm.at[wid, pl.ds((m + 1) * SB, SB)],
                                 src_i.at[nxt], isems[0])
                pltpu.async_copy(dst_hbm.at[wid, pl.ds((m + 1) * SB, SB)],
                                 dst_i.at[nxt], isems[1])

            for r in range(SB):
                buf = bufs[r % 2]
                gsem = gsems[r % 2]
                if r == SB - 2:
                    @pl.when(more)
                    def _():
                        pltpu.make_async_copy(
                            src_hbm.at[wid, pl.ds((m + 1) * SB, SB)],
                            src_i.at[nxt], isems[0]).wait()
                        pltpu.make_async_copy(
                            dst_hbm.at[wid, pl.ds((m + 1) * SB, SB)],
                            dst_i.at[nxt], isems[1]).wait()
                pltpu.make_async_copy(
                    hs_hbm.at[src_i.at[b, r]], buf, gsem).wait()
                pltpu.sync_copy(buf, acc.at[dst_i.at[b, r]], add=True)
                if r < SB - 2:
                    pltpu.async_copy(hs_hbm.at[src_i.at[b, r + 2]], buf, gsem)
                else:
                    @pl.when(more)
                    def _():
                        pltpu.async_copy(
                            hs_hbm.at[src_i.at[nxt, r + 2 - SB]], buf, gsem)

        def super_pair(p, carry):
            emit_super(p * 2, 0)
            emit_super(p * 2 + 1, 1)
            return carry

        lax.fori_loop(0, nsup // 2, super_pair, 0)
        plsc.subcore_barrier()
        pltpu.sync_copy(acc.at[pl.ds(sid * rpt, rpt)],
                        out_hbm.at[cid, pl.ds(sid * rpt, rpt)])

    return k


def _tc_a_body(deg_ref, x_ref, w_ref, hs_ref, dinv_ref):
    parts = deg_ref[...]
    deg = parts[0] + parts[1] + 1.0
    r = lax.rsqrt(deg)
    dinv = r * (1.5 - 0.5 * deg * r * r)
    h = jnp.dot(x_ref[...], w_ref[...],
                preferred_element_type=jnp.float32,
                precision=lax.Precision.DEFAULT)
    hs_ref[...] = h * dinv
    dinv_ref[...] = dinv


def _tc_b_body(t_ref, hs_ref, dinv_ref, b_ref, w_ref, out_ref):
    t = t_ref[0] + t_ref[1]
    dinv = dinv_ref[...]
    a = jnp.maximum(dinv * (t + hs_ref[...]) + b_ref[...], 0.0)
    out_ref[...] = jnp.dot(a, w_ref[...],
                           preferred_element_type=jnp.float32,
                           precision=lax.Precision.DEFAULT) * dinv


def _tc_c_body(t_ref, hs_ref, dinv_ref, b_ref, wout_ref, bout_ref, y_ref):
    t = t_ref[0] + t_ref[1]
    a = jnp.maximum(dinv_ref[...] * (t + hs_ref[...]) + b_ref[...], 0.0)
    y_ref[...] = jnp.dot(a, wout_ref[...],
                         preferred_element_type=jnp.float32,
                         precision=lax.Precision.DEFAULT) + bout_ref[...]


def kernel(x, edge_index, batch, W1, b1, W2, b2, Wout, bout):
    n = x.shape[0]
    e = edge_index.shape[1]
    assert D == x.shape[1]
    n_pad = -(-(n + 1) // 128) * 128

    nc = -(-e // (NW * C))
    nc = -(-nc // (2 * SB)) * (2 * SB)
    cap = NW * nc * C
    src = edge_index[0]
    dst = edge_index[1]
    srcp = jnp.concatenate(
        [src, jnp.zeros((cap - e,), jnp.int32)]).reshape(NW, nc, C)
    dstp = jnp.concatenate(
        [dst, jnp.full((cap - e,), n, jnp.int32)]).reshape(NW, nc, C)

    ones_blk = jnp.ones((C, D), jnp.float32)
    zeros_wide = jnp.zeros((n_pad, D), jnp.float32)

    deg_parts = _make_deg_kernel(n_pad, nc)(dstp, ones_blk, zeros_wide)

    B = 1000
    grid = (n // B,)
    row_block = lambda i: (i, 0)
    part_block = lambda i: (0, i, 0)
    fixed = lambda i: (0, 0)

    hs1, dinv = pl.pallas_call(
        _tc_a_body,
        grid=grid,
        in_specs=[
            pl.BlockSpec((NC, B, D), part_block),
            pl.BlockSpec((B, D), row_block),
            pl.BlockSpec((D, D), fixed),
        ],
        out_specs=[
            pl.BlockSpec((B, D), row_block),
            pl.BlockSpec((B, D), row_block),
        ],
        out_shape=[
            jax.ShapeDtypeStruct((n, D), jnp.float32),
            jax.ShapeDtypeStruct((n, D), jnp.float32),
        ],
    )(deg_parts, x, W1)

    scatter = _make_scatter_kernel(n_pad, nc)

    t1 = scatter(hs1, srcp, dstp, zeros_wide)

    hs2 = pl.pallas_call(
        _tc_b_body,
        grid=grid,
        in_specs=[
            pl.BlockSpec((NC, B, D), part_block),
            pl.BlockSpec((B, D), row_block),
            pl.BlockSpec((B, D), row_block),
            pl.BlockSpec((1, D), fixed),
            pl.BlockSpec((D, D), fixed),
        ],
        out_specs=pl.BlockSpec((B, D), row_block),
        out_shape=jax.ShapeDtypeStruct((n, D), jnp.float32),
    )(t1, hs1, dinv, b1.reshape(1, D), W2)

    t2 = scatter(hs2, srcp, dstp, zeros_wide)

    y = pl.pallas_call(
        _tc_c_body,
        grid=grid,
        in_specs=[
            pl.BlockSpec((NC, B, D), part_block),
            pl.BlockSpec((B, D), row_block),
            pl.BlockSpec((B, D), row_block),
            pl.BlockSpec((1, D), fixed),
            pl.BlockSpec((D, 1), fixed),
            pl.BlockSpec((1, 1), fixed),
        ],
        out_specs=pl.BlockSpec((B, 1), row_block),
        out_shape=jax.ShapeDtypeStruct((n, 1), jnp.float32),
    )(t2, hs2, dinv, b2.reshape(1, D), Wout, bout.reshape(1, 1))

    return y.reshape(-1)


# speedup vs baseline: 1.1326x; 1.1326x over previous
import functools

import jax
import jax.numpy as jnp
from jax import lax
from jax.experimental import pallas as pl
from jax.experimental.pallas import tpu as pltpu
from jax.experimental.pallas import tpu_sc as plsc

NC = 2
NS = 16
NW = NC * NS
C = 128
D = 128
SB = 8


def _make_deg_kernel(n_pad, nc):
    mesh = plsc.VectorSubcoreMesh(core_axis_name="c", subcore_axis_name="s")
    rpt = n_pad // NS

    @functools.partial(
        pl.kernel,
        mesh=mesh,
        out_type=jax.ShapeDtypeStruct((NC, n_pad, D), jnp.float32),
        scratch_types=[
            pltpu.VMEM((nc, C), jnp.int32),
            pltpu.VMEM((C, D), jnp.float32),
            pltpu.VMEM_SHARED((n_pad, D), jnp.float32),
        ],
    )
    def k(dst_hbm, ones_hbm, zeros_hbm, out_hbm, dst_v, ones_v, acc):
        cid = lax.axis_index("c")
        sid = lax.axis_index("s")
        wid = sid * NC + cid
        pltpu.sync_copy(zeros_hbm.at[pl.ds(sid * rpt, rpt)],
                        acc.at[pl.ds(sid * rpt, rpt)])
        pltpu.sync_copy(dst_hbm.at[wid], dst_v)
        pltpu.sync_copy(ones_hbm, ones_v)
        plsc.subcore_barrier()

        def body(j, carry):
            pltpu.sync_copy(ones_v, acc.at[dst_v.at[j]], add=True)
            return carry

        lax.fori_loop(0, nc, body, 0)
        plsc.subcore_barrier()
        pltpu.sync_copy(acc.at[pl.ds(sid * rpt, rpt)],
                        out_hbm.at[cid, pl.ds(sid * rpt, rpt)])

    return k


def _make_scatter_kernel(n_pad, nc):
    mesh = plsc.VectorSubcoreMesh(core_axis_name="c", subcore_axis_name="s")
    rpt = n_pad // NS
    nsup = nc // SB

    @functools.partial(
        pl.kernel,
        mesh=mesh,
        out_type=jax.ShapeDtypeStruct((NC, n_pad, D), jnp.float32),
        scratch_types=[
            pltpu.VMEM((2, SB, C), jnp.int32),
            pltpu.VMEM((2, SB, C), jnp.int32),
            pltpu.VMEM((C, D), jnp.float32),
            pltpu.VMEM((C, D), jnp.float32),
            pltpu.VMEM_SHARED((n_pad, D), jnp.float32),
            pltpu.SemaphoreType.DMA,
            pltpu.SemaphoreType.DMA,
            pltpu.SemaphoreType.DMA,
            pltpu.SemaphoreType.DMA,
        ],
    )
    def k(hs_hbm, src_hbm, dst_hbm, zeros_hbm, out_hbm,
          src_i, dst_i, buf0, buf1, acc, g0, g1, is0, is1):
        cid = lax.axis_index("c")
        sid = lax.axis_index("s")
        wid = sid * NC + cid
        hsc = hs_hbm.at[cid]
        pltpu.sync_copy(zeros_hbm.at[pl.ds(sid * rpt, rpt)],
                        acc.at[pl.ds(sid * rpt, rpt)])
        pltpu.sync_copy(src_hbm.at[wid, pl.ds(0, SB)], src_i.at[0])
        pltpu.sync_copy(dst_hbm.at[wid, pl.ds(0, SB)], dst_i.at[0])
        plsc.subcore_barrier()

        bufs = (buf0, buf1)
        gsems = (g0, g1)
        isems = (is0, is1)

        pltpu.async_copy(hsc.at[src_i.at[0, 0]], buf0, g0)
        pltpu.async_copy(hsc.at[src_i.at[0, 1]], buf1, g1)

        def emit_super(m, b):
            nxt = 1 - b
            more = m + 1 < nsup

            @pl.when(more)
            def _():
                pltpu.async_copy(src_hbm.at[wid, pl.ds((m + 1) * SB, SB)],
                                 src_i.at[nxt], isems[0])
                pltpu.async_copy(dst_hbm.at[wid, pl.ds((m + 1) * SB, SB)],
                                 dst_i.at[nxt], isems[1])

            for r in range(SB):
                buf = bufs[r % 2]
                gsem = gsems[r % 2]
                if r == SB - 2:
                    @pl.when(more)
                    def _():
                        pltpu.make_async_copy(
                            src_hbm.at[wid, pl.ds((m + 1) * SB, SB)],
                            src_i.at[nxt], isems[0]).wait()
                        pltpu.make_async_copy(
                            dst_hbm.at[wid, pl.ds((m + 1) * SB, SB)],
                            dst_i.at[nxt], isems[1]).wait()
                pltpu.make_async_copy(
                    hsc.at[src_i.at[b, r]], buf, gsem).wait()
                pltpu.sync_copy(buf, acc.at[dst_i.at[b, r]], add=True)
                if r < SB - 2:
                    pltpu.async_copy(hsc.at[src_i.at[b, r + 2]], buf, gsem)
                else:
                    @pl.when(more)
                    def _():
                        pltpu.async_copy(
                            hsc.at[src_i.at[nxt, r + 2 - SB]], buf, gsem)

        def super_pair(p, carry):
            emit_super(p * 2, 0)
            emit_super(p * 2 + 1, 1)
            return carry

        lax.fori_loop(0, nsup // 2, super_pair, 0)
        plsc.subcore_barrier()
        pltpu.sync_copy(acc.at[pl.ds(sid * rpt, rpt)],
                        out_hbm.at[cid, pl.ds(sid * rpt, rpt)])

    return k


def _tc_a_body(deg_ref, x_ref, w_ref, hs_ref, dinv_ref):
    parts = deg_ref[...]
    deg = parts[0] + parts[1] + 1.0
    r = lax.rsqrt(deg)
    dinv = r * (1.5 - 0.5 * deg * r * r)
    h = jnp.dot(x_ref[...], w_ref[...],
                preferred_element_type=jnp.float32,
                precision=lax.Precision.DEFAULT)
    hs = h * dinv
    hs_ref[...] = jnp.broadcast_to(hs[None], (NC,) + hs.shape)
    dinv_ref[...] = dinv


def _tc_b_body(t_ref, hs_ref, dinv_ref, b_ref, w_ref, out_ref):
    t = t_ref[0] + t_ref[1]
    dinv = dinv_ref[...]
    a = jnp.maximum(dinv * (t + hs_ref[0]) + b_ref[...], 0.0)
    hs = jnp.dot(a, w_ref[...],
                 preferred_element_type=jnp.float32,
                 precision=lax.Precision.DEFAULT) * dinv
    out_ref[...] = jnp.broadcast_to(hs[None], (NC,) + hs.shape)


def _tc_c_body(t_ref, hs_ref, dinv_ref, b_ref, wout_ref, bout_ref, y_ref):
    t = t_ref[0] + t_ref[1]
    a = jnp.maximum(dinv_ref[...] * (t + hs_ref[0]) + b_ref[...], 0.0)
    y_ref[...] = jnp.dot(a, wout_ref[...],
                         preferred_element_type=jnp.float32,
                         precision=lax.Precision.DEFAULT) + bout_ref[...]


def kernel(x, edge_index, batch, W1, b1, W2, b2, Wout, bout):
    n = x.shape[0]
    e = edge_index.shape[1]
    assert D == x.shape[1]
    n_pad = -(-(n + 1) // 128) * 128

    nc = -(-e // (NW * C))
    nc = -(-nc // (2 * SB)) * (2 * SB)
    cap = NW * nc * C
    src = edge_index[0]
    dst = edge_index[1]
    srcp = jnp.concatenate(
        [src, jnp.zeros((cap - e,), jnp.int32)]).reshape(NW, nc, C)
    dstp = jnp.concatenate(
        [dst, jnp.full((cap - e,), n, jnp.int32)]).reshape(NW, nc, C)

    ones_blk = jnp.ones((C, D), jnp.float32)
    zeros_wide = jnp.zeros((n_pad, D), jnp.float32)

    deg_parts = _make_deg_kernel(n_pad, nc)(dstp, ones_blk, zeros_wide)

    B = 1000
    grid = (n // B,)
    row_block = lambda i: (i, 0)
    part_block = lambda i: (0, i, 0)
    fixed = lambda i: (0, 0)

    hs1, dinv = pl.pallas_call(
        _tc_a_body,
        grid=grid,
        in_specs=[
            pl.BlockSpec((NC, B, D), part_block),
            pl.BlockSpec((B, D), row_block),
            pl.BlockSpec((D, D), fixed),
        ],
        out_specs=[
            pl.BlockSpec((NC, B, D), part_block),
            pl.BlockSpec((B, D), row_block),
        ],
        out_shape=[
            jax.ShapeDtypeStruct((NC, n, D), jnp.float32),
            jax.ShapeDtypeStruct((n, D), jnp.float32),
        ],
    )(deg_parts, x, W1)

    scatter = _make_scatter_kernel(n_pad, nc)

    t1 = scatter(hs1, srcp, dstp, zeros_wide)

    hs2 = pl.pallas_call(
        _tc_b_body,
        grid=grid,
        in_specs=[
            pl.BlockSpec((NC, B, D), part_block),
            pl.BlockSpec((NC, B, D), part_block),
            pl.BlockSpec((B, D), row_block),
            pl.BlockSpec((1, D), fixed),
            pl.BlockSpec((D, D), fixed),
        ],
        out_specs=pl.BlockSpec((NC, B, D), part_block),
        out_shape=jax.ShapeDtypeStruct((NC, n, D), jnp.float32),
    )(t1, hs1, dinv, b1.reshape(1, D), W2)

    t2 = scatter(hs2, srcp, dstp, zeros_wide)

    y = pl.pallas_call(
        _tc_c_body,
        grid=grid,
        in_specs=[
            pl.BlockSpec((NC, B, D), part_block),
            pl.BlockSpec((NC, B, D), part_block),
            pl.BlockSpec((B, D), row_block),
            pl.BlockSpec((1, D), fixed),
            pl.BlockSpec((D, 1), fixed),
            pl.BlockSpec((1, 1), fixed),
        ],
        out_specs=pl.BlockSpec((B, 1), row_block),
        out_shape=jax.ShapeDtypeStruct((n, 1), jnp.float32),
    )(t2, hs2, dinv, b2.reshape(1, D), Wout, bout.reshape(1, 1))

    return y.reshape(-1)
